# baseline (speedup 1.0000x reference)
import functools

import jax
import jax.numpy as jnp
from jax import lax
from jax.experimental import pallas as pl
from jax.experimental.pallas import tpu as pltpu
from jax.experimental.pallas import tpu_sc as plsc

THR_CONF = 0.5
THR_NMS = 0.45
NUM_CLASSES = 20
MAX_DET = 100
B = 16
A = 10647
A_PAD = 10752
OUT_PAD = 112
L = 16
STEPS = A_PAD // L
BLK = 2688


def _decode_body(pt_ref, anc_ref, fs_ref,
                 l_ref, t_ref, r_ref, b_ref,
                 l0_ref, t0_ref, r0_ref, b0_ref,
                 sc_ref, lab_ref):
    cf = pt_ref[4]
    conf = jax.nn.sigmoid(cf)
    sc_ref[...] = jnp.where(conf > THR_CONF, conf, 0.0)

    best = jax.nn.sigmoid(pt_ref[5])
    labv = jnp.ones(best.shape, jnp.int32)
    for c in range(1, NUM_CLASSES):
        v = jax.nn.sigmoid(pt_ref[5 + c])
        upd = v > best
        best = jnp.where(upd, v, best)
        labv = jnp.where(upd, c + 1, labv)
    lab_ref[...] = labv

    xyx = jax.nn.sigmoid(pt_ref[0]) / fs_ref[0:1] + anc_ref[0:1]
    xyy = jax.nn.sigmoid(pt_ref[1]) / fs_ref[1:2] + anc_ref[1:2]
    whx = jnp.exp(pt_ref[2]) * anc_ref[2:3]
    why = jnp.exp(pt_ref[3]) * anc_ref[3:4]
    l0 = xyx - whx * 0.5
    t0 = xyy - why * 0.5
    r0 = xyx + whx * 0.5
    b0 = xyy + why * 0.5
    l0_ref[...] = l0
    t0_ref[...] = t0
    r0_ref[...] = r0
    b0_ref[...] = b0
    off = labv.astype(jnp.float32) * 1000.0
    l_ref[...] = l0 + off
    t_ref[...] = t0 + off
    r_ref[...] = r0 + off
    b_ref[...] = b0 + off


def _decode(pt, anct, fst):
    f32 = jnp.float32
    outs = [jax.ShapeDtypeStruct((B, A_PAD), f32)] * 9 + [
        jax.ShapeDtypeStruct((B, A_PAD), jnp.int32)
    ]
    outs = outs[:8] + [jax.ShapeDtypeStruct((B, A_PAD), f32),
                       jax.ShapeDtypeStruct((B, A_PAD), jnp.int32)]
    grid = (A_PAD // BLK,)
    obs = pl.BlockSpec((B, BLK), lambda i: (0, i))
    return pl.pallas_call(
        _decode_body,
        grid=grid,
        in_specs=[
            pl.BlockSpec((5 + NUM_CLASSES, B, BLK), lambda i: (0, 0, i)),
            pl.BlockSpec((4, BLK), lambda i: (0, i)),
            pl.BlockSpec((2, BLK), lambda i: (0, i)),
        ],
        out_specs=[obs] * 10,
        out_shape=outs,
    )(pt, anct, fst)


def _nms_body(l_h, t_h, r_h, b_h, l0_h, t0_h, r0_h, b0_h, sc_h, lab_h,
              obl_h, obt_h, obr_h, obb_h, osc_h, olab_h, oid_h,
              l_v, t_v, r_v, b_v, l0_v, t0_v, r0_v, b0_v, sc_v, lab_v,
              obl_v, obt_v, obr_v, obb_v, osc_v, olab_v, oid_v):
    cid = lax.axis_index("c")
    sid = lax.axis_index("s")

    @pl.when(cid == 0)
    def _():
        bi = sid
        pltpu.sync_copy(l_h.at[bi], l_v)
        pltpu.sync_copy(t_h.at[bi], t_v)
        pltpu.sync_copy(r_h.at[bi], r_v)
        pltpu.sync_copy(b_h.at[bi], b_v)
        pltpu.sync_copy(l0_h.at[bi], l0_v)
        pltpu.sync_copy(t0_h.at[bi], t0_v)
        pltpu.sync_copy(r0_h.at[bi], r0_v)
        pltpu.sync_copy(b0_h.at[bi], b0_v)
        pltpu.sync_copy(sc_h.at[bi], sc_v)
        pltpu.sync_copy(lab_h.at[bi], lab_v)

        zf = jnp.zeros((L,), jnp.float32)
        zi = jnp.zeros((L,), jnp.int32)
        lane = lax.iota(jnp.int32, L)
        big = jnp.full((L,), 2**30, jnp.int32)

        def arg_body(j, carry):
            m, idxv = carry
            base = j * L
            sv = sc_v[pl.ds(base, L)]
            posv = base + lane
            upd = sv > m
            return jnp.where(upd, sv, m), jnp.where(upd, posv, idxv)

        m0, i0 = lax.fori_loop(0, STEPS, arg_body, (zf, zi))

        def body(k, carry):
            m, idxv = carry
            mv = m
            for sh in (8, 4, 2, 1):
                mv = jnp.maximum(mv, jnp.take(mv, lane ^ sh))
            iv = jnp.where(m == mv, idxv, big)
            for sh in (8, 4, 2, 1):
                iv = jnp.minimum(iv, jnp.take(iv, lane ^ sh))
            go_v = mv > 0.0
            i_star = iv[0]
            isl = pl.ds(i_star, L)
            osl = pl.ds(k * L, L)
            lsv = jnp.full((L,), l_v[isl][0])
            tsv = jnp.full((L,), t_v[isl][0])
            rsv = jnp.full((L,), r_v[isl][0])
            bsv = jnp.full((L,), b_v[isl][0])
            a1v = (rsv - lsv) * (bsv - tsv)
            obl_v[osl] = jnp.where(go_v, jnp.full((L,), l0_v[isl][0]), 0.0)
            obt_v[osl] = jnp.where(go_v, jnp.full((L,), t0_v[isl][0]), 0.0)
            obr_v[osl] = jnp.where(go_v, jnp.full((L,), r0_v[isl][0]), 0.0)
            obb_v[osl] = jnp.where(go_v, jnp.full((L,), b0_v[isl][0]), 0.0)
            osc_v[osl] = jnp.where(go_v, mv, 0.0)
            olab_v[osl] = jnp.where(go_v, jnp.full((L,), lab_v[isl][0]), 0)
            oid_v[osl] = jnp.where(go_v, jnp.full((L,), bi, jnp.int32), -1)

            def sweep(j, carry2):
                m2, idx2 = carry2
                base = j * L
                sl = pl.ds(base, L)
                sv = sc_v[sl]
                lv = l_v[sl]
                tv = t_v[sl]
                rv = r_v[sl]
                bv = b_v[sl]
                ltx = jnp.maximum(lsv, lv)
                lty = jnp.maximum(tsv, tv)
                rbx = jnp.minimum(rsv, rv)
                rby = jnp.minimum(bsv, bv)
                wx = jnp.maximum(rbx - ltx, 0.0)
                wy = jnp.maximum(rby - lty, 0.0)
                inter = wx * wy
                a2 = (rv - lv) * (bv - tv)
                iou = inter / (a1v + a2 - inter + 1e-9)
                posv = base + lane
                kill = (iou > THR_NMS) | (posv == iv)
                sv2 = jnp.where(kill, 0.0, sv)
                sc_v[sl] = sv2
                upd = sv2 > m2
                return jnp.where(upd, sv2, m2), jnp.where(upd, posv, idx2)

            return lax.fori_loop(0, STEPS, sweep, (zf, zi))

        lax.fori_loop(0, MAX_DET, body, (m0, i0))

        pltpu.sync_copy(obl_v, obl_h.at[bi])
        pltpu.sync_copy(obt_v, obt_h.at[bi])
        pltpu.sync_copy(obr_v, obr_h.at[bi])
        pltpu.sync_copy(obb_v, obb_h.at[bi])
        pltpu.sync_copy(osc_v, osc_h.at[bi])
        pltpu.sync_copy(olab_v, olab_h.at[bi])
        pltpu.sync_copy(oid_v, oid_h.at[bi])


def _nms(l, t, r, b, l0, t0, r0, b0, sc, lab):
    f32 = jnp.float32
    i32 = jnp.int32
    mesh = plsc.VectorSubcoreMesh(core_axis_name="c", subcore_axis_name="s")
    out_type = [jax.ShapeDtypeStruct((B, MAX_DET * L), f32)] * 5 + [
        jax.ShapeDtypeStruct((B, MAX_DET * L), i32),
        jax.ShapeDtypeStruct((B, MAX_DET * L), i32),
    ]
    scratch = (
        [pltpu.VMEM((A_PAD,), f32)] * 9
        + [pltpu.VMEM((A_PAD,), i32)]
        + [pltpu.VMEM((MAX_DET * L,), f32)] * 5
        + [pltpu.VMEM((MAX_DET * L,), i32)] * 2
    )
    fn = pl.kernel(_nms_body, out_type=out_type, mesh=mesh,
                   scratch_types=scratch)
    return fn(l, t, r, b, l0, t0, r0, b0, sc, lab)


def kernel(p_yolo_ts4, ancs, fsize_p):
    pad = A_PAD - A
    pt = jnp.transpose(
        jnp.pad(p_yolo_ts4, ((0, 0), (0, pad), (0, 0))), (2, 0, 1))
    anct = jnp.transpose(
        jnp.pad(ancs, ((0, pad), (0, 0)), constant_values=0.25), (1, 0))
    fst = jnp.transpose(
        jnp.pad(fsize_p, ((0, pad), (0, 0)), constant_values=1.0), (1, 0))

    l, t, r, b, l0, t0, r0, b0, sc, lab = _decode(pt, anct, fst)
    obl, obt, obr, obb, osc, olab, oid = _nms(
        l, t, r, b, l0, t0, r0, b0, sc, lab)

    obl, obt, obr, obb, osc, olab, oid = (
        a.reshape(B, MAX_DET, L)[:, :, 0]
        for a in (obl, obt, obr, obb, osc, olab, oid))
    ids = oid.reshape(-1)
    boxes = jnp.stack([obl, obt, obr, obb], axis=-1).reshape(-1, 4)
    labels = olab.reshape(-1)
    scores = osc.reshape(-1)
    return ids, boxes, labels, scores
